# Initial kernel scaffold; baseline (speedup 1.0000x reference)
import functools

import jax
import jax.numpy as jnp
from jax import lax
from jax.experimental import pallas as pl
from jax.experimental.pallas import tpu as pltpu
from jax.experimental.pallas import tpu_sc as plsc

_N = 10000
_E = 320000
_D = 128
_CL = 16
_NC = 2
_NS = 16
_NW = _NC * _NS
_EPW = _E // _NW
_CH = 80
_NCHUNK = _EPW // _CH
_RPS = _N // _NS
_HW = 16


def _mesh():
    return plsc.VectorSubcoreMesh(core_axis_name="c", subcore_axis_name="s")


@functools.partial(
    pl.kernel,
    out_type=jax.ShapeDtypeStruct((_NC, 2, _N, _HW), jnp.float32),
    mesh=_mesh(),
    scratch_types=[
        pltpu.VMEM((_NCHUNK, _CH), jnp.int32),
        pltpu.VMEM((_NCHUNK, _CH), jnp.int32),
        pltpu.VMEM((_CH,), jnp.int32),
        pltpu.VMEM((_CH,), jnp.int32),
        pltpu.VMEM((_CH, _HW), jnp.float32),
        pltpu.VMEM_SHARED((_N, _HW), jnp.float32),
        pltpu.VMEM_SHARED((_N, _HW), jnp.float32),
    ],
)
def _hist_kernel(src_hbm, dst_hbm, ones_hbm, zeros_hbm, out_hbm,
                 srcb, dstb, t0, t1, onesb, h0, h1):
    cid = lax.axis_index("c")
    sid = lax.axis_index("s")
    wid = cid * _NS + sid
    pltpu.sync_copy(src_hbm.at[wid], srcb)
    pltpu.sync_copy(dst_hbm.at[wid], dstb)
    pltpu.sync_copy(ones_hbm, onesb)
    r0 = sid * _RPS
    pltpu.sync_copy(zeros_hbm, h0.at[pl.ds(r0, _RPS)])
    pltpu.sync_copy(zeros_hbm, h1.at[pl.ds(r0, _RPS)])
    plsc.subcore_barrier()

    def body(j, carry):
        for k in range(_CH // 16):
            t0[pl.ds(k * 16, 16)] = srcb[j, pl.ds(k * 16, 16)]
            t1[pl.ds(k * 16, 16)] = dstb[j, pl.ds(k * 16, 16)]
        pltpu.sync_copy(onesb, h0.at[t0], add=True)
        pltpu.sync_copy(onesb, h1.at[t1], add=True)
        return carry

    lax.fori_loop(0, _NCHUNK, body, 0)
    plsc.subcore_barrier()
    pltpu.sync_copy(h0.at[pl.ds(r0, _RPS)], out_hbm.at[cid, 0, pl.ds(r0, _RPS)])
    pltpu.sync_copy(h1.at[pl.ds(r0, _RPS)], out_hbm.at[cid, 1, pl.ds(r0, _RPS)])


def _make_scatter_kernel(d):
    @functools.partial(
        pl.kernel,
        out_type=jax.ShapeDtypeStruct((_NC, _N, d), jnp.float32),
        mesh=_mesh(),
        scratch_types=[
            pltpu.VMEM((_NCHUNK, _CH), jnp.int32),
            pltpu.VMEM((_NCHUNK, _CH), jnp.int32),
            pltpu.VMEM((_CH,), jnp.int32),
            pltpu.VMEM((_CH, d), jnp.float32),
            pltpu.VMEM((_CH, d), jnp.float32),
            pltpu.VMEM_SHARED((_N, d), jnp.float32),
            pltpu.SemaphoreType.DMA,
            pltpu.SemaphoreType.DMA,
        ],
    )
    def scat(z_hbm, src_hbm, dst_hbm, zeros_hbm, out_hbm,
             srcb, dstb, tdst, rows0, rows1, acc, sem0, sem1):
        cid = lax.axis_index("c")
        sid = lax.axis_index("s")
        wid = cid * _NS + sid
        pltpu.sync_copy(src_hbm.at[wid], srcb)
        pltpu.sync_copy(dst_hbm.at[wid], dstb)
        r0 = sid * _RPS
        pltpu.sync_copy(zeros_hbm, acc.at[pl.ds(r0, _RPS)])
        plsc.subcore_barrier()

        pltpu.async_copy(z_hbm.at[srcb.at[0]], rows0, sem0)

        def chunk(j, rows_cur, sem_cur, rows_nxt, sem_nxt):
            @pl.when(j + 1 < _NCHUNK)
            def _start_next():
                pltpu.async_copy(z_hbm.at[srcb.at[j + 1]], rows_nxt, sem_nxt)

            for k in range(_CH // 16):
                tdst[pl.ds(k * 16, 16)] = dstb[j, pl.ds(k * 16, 16)]
            pltpu.make_async_copy(z_hbm.at[srcb.at[j]], rows_cur, sem_cur).wait()
            pltpu.sync_copy(rows_cur, acc.at[tdst], add=True)

        def body(j, carry):
            @pl.when(j % 2 == 0)
            def _even():
                chunk(j, rows0, sem0, rows1, sem1)

            @pl.when(j % 2 == 1)
            def _odd():
                chunk(j, rows1, sem1, rows0, sem0)

            return carry

        lax.fori_loop(0, _NCHUNK, body, 0)
        plsc.subcore_barrier()
        pltpu.sync_copy(acc.at[pl.ds(r0, _RPS)], out_hbm.at[cid, pl.ds(r0, _RPS)])

    return scat


_scatter_d = _make_scatter_kernel(_D)
_scatter_c = _make_scatter_kernel(_CL)


def _tc1_body(h_ref, w_ref, dp_ref, z_ref, ac_ref):
    dp = dp_ref[...]
    deg_out = dp[:, 0:1] + dp[:, 1:2]
    deg_in = dp[:, 2:3] + dp[:, 3:4]
    a = lax.rsqrt(jnp.maximum(deg_out, 1.0))
    c = lax.rsqrt(jnp.maximum(deg_in, 1.0))
    y = jnp.dot(h_ref[...], w_ref[...], preferred_element_type=jnp.float32)
    z_ref[...] = y * a
    ac_ref[...] = jnp.concatenate([a, c], axis=1)


def _tc2_body(p_ref, ac_ref, b1_ref, w2_ref, z2_ref):
    a = ac_ref[:, 0:1]
    c = ac_ref[:, 1:2]
    agg = p_ref[0] + p_ref[1]
    h1 = jnp.maximum(agg * c + b1_ref[...], 0.0)
    z2_ref[...] = jnp.dot(h1, w2_ref[...], preferred_element_type=jnp.float32) * a


def _tc3_body(q_ref, ac_ref, b2_ref, o_ref):
    c = ac_ref[:, 1:2]
    o_ref[...] = (q_ref[0] + q_ref[1]) * c + b2_ref[...]


def kernel(h, edge_index, W1, b1, W2, b2):
    src = edge_index[0].reshape(_NW, _NCHUNK, _CH)
    dst = edge_index[1].reshape(_NW, _NCHUNK, _CH)
    ones_rows = jnp.ones((_CH, _HW), jnp.float32)
    zeros_h = jnp.zeros((_RPS, _HW), jnp.float32)
    zeros_d = jnp.zeros((_RPS, _D), jnp.float32)
    zeros_c = jnp.zeros((_RPS, _CL), jnp.float32)

    hist = _hist_kernel(src, dst, ones_rows, zeros_h)
    dp = hist[:, :, :, 0].reshape(4, _N).T

    z1, ac = pl.pallas_call(
        _tc1_body,
        out_shape=[
            jax.ShapeDtypeStruct((_N, _D), jnp.float32),
            jax.ShapeDtypeStruct((_N, 2), jnp.float32),
        ],
    )(h, W1, dp)

    p = _scatter_d(z1, src, dst, zeros_d)

    z2 = pl.pallas_call(
        _tc2_body,
        out_shape=jax.ShapeDtypeStruct((_N, _CL), jnp.float32),
    )(p, ac, b1.reshape(1, _D), W2)

    q = _scatter_c(z2, src, dst, zeros_c)

    out = pl.pallas_call(
        _tc3_body,
        out_shape=jax.ShapeDtypeStruct((_N, _CL), jnp.float32),
    )(q, ac, b2.reshape(1, _CL))
    return out


# speedup vs baseline: 16.9845x; 16.9845x over previous
import functools

import jax
import jax.numpy as jnp
from jax import lax
from jax.experimental import pallas as pl
from jax.experimental.pallas import tpu as pltpu
from jax.experimental.pallas import tpu_sc as plsc

_N = 10000
_E = 320000
_D = 128
_CL = 16
_NC = 2
_NS = 16
_NW = _NC * _NS
_EPW = _E // _NW
_CH = 80
_NCHUNK = _EPW // _CH
_NP = 10240
_RPS = _NP // _NS
_HW = 16


def _mesh():
    return plsc.VectorSubcoreMesh(core_axis_name="c", subcore_axis_name="s")


@functools.partial(
    pl.kernel,
    out_type=jax.ShapeDtypeStruct((_NC, 2, _NP, _HW), jnp.float32),
    mesh=_mesh(),
    compiler_params=pltpu.CompilerParams(use_tc_tiling_on_sc=False),
    scratch_types=[
        pltpu.VMEM((_CH,), jnp.int32),
        pltpu.VMEM((_CH,), jnp.int32),
        pltpu.VMEM((_CH, _HW), jnp.float32),
        pltpu.VMEM_SHARED((_NP, _HW), jnp.float32),
        pltpu.VMEM_SHARED((_NP, _HW), jnp.float32),
    ],
)
def _hist_kernel(src_hbm, dst_hbm, ones_hbm, zeros_hbm, out_hbm,
                 t0, t1, onesb, h0, h1):
    cid = lax.axis_index("c")
    sid = lax.axis_index("s")
    wid = cid * _NS + sid
    pltpu.sync_copy(ones_hbm, onesb)
    r0 = sid * _RPS
    pltpu.sync_copy(zeros_hbm, h0.at[pl.ds(r0, _RPS)])
    pltpu.sync_copy(zeros_hbm, h1.at[pl.ds(r0, _RPS)])
    plsc.subcore_barrier()

    def body(j, carry):
        pltpu.sync_copy(src_hbm.at[wid, j], t0)
        pltpu.sync_copy(dst_hbm.at[wid, j], t1)
        pltpu.sync_copy(onesb, h0.at[t0], add=True)
        pltpu.sync_copy(onesb, h1.at[t1], add=True)
        return carry

    lax.fori_loop(0, _NCHUNK, body, 0)
    plsc.subcore_barrier()
    pltpu.sync_copy(h0.at[pl.ds(r0, _RPS)], out_hbm.at[cid, 0, pl.ds(r0, _RPS)])
    pltpu.sync_copy(h1.at[pl.ds(r0, _RPS)], out_hbm.at[cid, 1, pl.ds(r0, _RPS)])


_DH = _D // _NC
_EPS = _E // _NS
_NCHUNK2 = _EPS // _CH


@functools.partial(
    pl.kernel,
    out_type=jax.ShapeDtypeStruct((_NC, _NP, _DH), jnp.float32),
    mesh=_mesh(),
    compiler_params=pltpu.CompilerParams(use_tc_tiling_on_sc=False),
    scratch_types=[
        pltpu.VMEM((_NCHUNK2, _CH), jnp.int32),
        pltpu.VMEM((_NCHUNK2, _CH), jnp.int32),
        pltpu.VMEM((_CH,), jnp.int32),
        pltpu.VMEM((_CH, _DH), jnp.float32),
        pltpu.VMEM((_CH, _DH), jnp.float32),
        pltpu.VMEM_SHARED((_NP, _DH), jnp.float32),
        pltpu.SemaphoreType.DMA,
        pltpu.SemaphoreType.DMA,
    ],
)
def _scatter_d(z_hbm, src_hbm, dst_hbm, zeros_hbm, out_hbm,
               srcb, dstb, tdst, rows0, rows1, acc, sem0, sem1):
    cid = lax.axis_index("c")
    sid = lax.axis_index("s")
    pltpu.sync_copy(src_hbm.at[sid], srcb)
    pltpu.sync_copy(dst_hbm.at[sid], dstb)
    r0 = sid * _RPS
    pltpu.sync_copy(zeros_hbm, acc.at[pl.ds(r0, _RPS)])
    plsc.subcore_barrier()

    zc = z_hbm.at[cid]
    pltpu.async_copy(zc.at[srcb.at[0]], rows0, sem0)

    def chunk(j, rows_cur, sem_cur, rows_nxt, sem_nxt):
        @pl.when(j + 1 < _NCHUNK2)
        def _start_next():
            pltpu.async_copy(zc.at[srcb.at[j + 1]], rows_nxt, sem_nxt)

        for k in range(_CH // 16):
            tdst[pl.ds(k * 16, 16)] = dstb[j, pl.ds(k * 16, 16)]
        pltpu.make_async_copy(zc.at[srcb.at[j]], rows_cur, sem_cur).wait()
        pltpu.sync_copy(rows_cur, acc.at[tdst], add=True)

    def body(j, carry):
        @pl.when(j % 2 == 0)
        def _even():
            chunk(j, rows0, sem0, rows1, sem1)

        @pl.when(j % 2 == 1)
        def _odd():
            chunk(j, rows1, sem1, rows0, sem0)

        return carry

    lax.fori_loop(0, _NCHUNK2, body, 0)
    plsc.subcore_barrier()
    pltpu.sync_copy(acc.at[pl.ds(r0, _RPS)], out_hbm.at[cid, pl.ds(r0, _RPS)])


@functools.partial(
    pl.kernel,
    out_type=jax.ShapeDtypeStruct((_NC, _NP, _CL), jnp.float32),
    mesh=_mesh(),
    compiler_params=pltpu.CompilerParams(use_tc_tiling_on_sc=False),
    scratch_types=[
        pltpu.VMEM((_NCHUNK, _CH), jnp.int32),
        pltpu.VMEM((_NCHUNK, _CH), jnp.int32),
        pltpu.VMEM((_CH,), jnp.int32),
        pltpu.VMEM((_CH, _CL), jnp.float32),
        pltpu.VMEM((_CH, _CL), jnp.float32),
        pltpu.VMEM_SHARED((_NP, _CL), jnp.float32),
        pltpu.SemaphoreType.DMA,
        pltpu.SemaphoreType.DMA,
    ],
)
def _scatter_c(z_hbm, src_hbm, dst_hbm, zeros_hbm, out_hbm,
               srcb, dstb, tdst, rows0, rows1, acc, sem0, sem1):
    cid = lax.axis_index("c")
    sid = lax.axis_index("s")
    wid = cid * _NS + sid
    pltpu.sync_copy(src_hbm.at[wid], srcb)
    pltpu.sync_copy(dst_hbm.at[wid], dstb)
    r0 = sid * _RPS
    pltpu.sync_copy(zeros_hbm, acc.at[pl.ds(r0, _RPS)])
    plsc.subcore_barrier()

    pltpu.async_copy(z_hbm.at[srcb.at[0]], rows0, sem0)

    def chunk(j, rows_cur, sem_cur, rows_nxt, sem_nxt):
        @pl.when(j + 1 < _NCHUNK)
        def _start_next():
            pltpu.async_copy(z_hbm.at[srcb.at[j + 1]], rows_nxt, sem_nxt)

        for k in range(_CH // 16):
            tdst[pl.ds(k * 16, 16)] = dstb[j, pl.ds(k * 16, 16)]
        pltpu.make_async_copy(z_hbm.at[srcb.at[j]], rows_cur, sem_cur).wait()
        pltpu.sync_copy(rows_cur, acc.at[tdst], add=True)

    def body(j, carry):
        @pl.when(j % 2 == 0)
        def _even():
            chunk(j, rows0, sem0, rows1, sem1)

        @pl.when(j % 2 == 1)
        def _odd():
            chunk(j, rows1, sem1, rows0, sem0)

        return carry

    lax.fori_loop(0, _NCHUNK, body, 0)
    plsc.subcore_barrier()
    pltpu.sync_copy(acc.at[pl.ds(r0, _RPS)], out_hbm.at[cid, pl.ds(r0, _RPS)])


def _tc1_body(h_ref, w_ref, dp_ref, z_ref, ac_ref):
    dp = dp_ref[...]
    deg_out = dp[:, 0:1] + dp[:, 2:3]
    deg_in = dp[:, 1:2] + dp[:, 3:4]
    a = lax.rsqrt(jnp.maximum(deg_out, 1.0))
    c = lax.rsqrt(jnp.maximum(deg_in, 1.0))
    y = jnp.dot(h_ref[...], w_ref[...], preferred_element_type=jnp.float32)
    z_ref[...] = y * a
    ac_ref[...] = jnp.concatenate([a, c], axis=1)


def _tc2_body(p_ref, ac_ref, b1_ref, w2_ref, z2_ref):
    a = ac_ref[:, 0:1]
    c = ac_ref[:, 1:2]
    agg = jnp.concatenate([p_ref[0], p_ref[1]], axis=1)
    h1 = jnp.maximum(agg * c + b1_ref[...], 0.0)
    z2_ref[...] = jnp.dot(h1, w2_ref[...], preferred_element_type=jnp.float32) * a


def _tc3_body(q_ref, ac_ref, b2_ref, o_ref):
    c = ac_ref[:, 1:2]
    o_ref[...] = (q_ref[0] + q_ref[1]) * c + b2_ref[...]


def kernel(h, edge_index, W1, b1, W2, b2):
    src32 = edge_index[0].reshape(_NW, _NCHUNK, _CH)
    dst32 = edge_index[1].reshape(_NW, _NCHUNK, _CH)
    src16 = edge_index[0].reshape(_NS, _NCHUNK2, _CH)
    dst16 = edge_index[1].reshape(_NS, _NCHUNK2, _CH)
    ones_rows = jnp.ones((_CH, _HW), jnp.float32)
    zeros_h = jnp.zeros((_RPS, _HW), jnp.float32)
    zeros_d = jnp.zeros((_RPS, _DH), jnp.float32)
    zeros_c = jnp.zeros((_RPS, _CL), jnp.float32)

    hist = _hist_kernel(src32, dst32, ones_rows, zeros_h)
    dp = hist[:, :, :_N, 0].reshape(4, _N).T

    z1, ac = pl.pallas_call(
        _tc1_body,
        out_shape=[
            jax.ShapeDtypeStruct((_N, _D), jnp.float32),
            jax.ShapeDtypeStruct((_N, 2), jnp.float32),
        ],
    )(h, W1, dp)

    z1s = z1.reshape(_N, _NC, _DH).transpose(1, 0, 2)
    p = _scatter_d(z1s, src16, dst16, zeros_d)[:, :_N]

    z2 = pl.pallas_call(
        _tc2_body,
        out_shape=jax.ShapeDtypeStruct((_N, _CL), jnp.float32),
    )(p, ac, b1.reshape(1, _D), W2)

    q = _scatter_c(z2, src32, dst32, zeros_c)[:, :_N]

    out = pl.pallas_call(
        _tc3_body,
        out_shape=jax.ShapeDtypeStruct((_N, _CL), jnp.float32),
    )(q, ac, b2.reshape(1, _CL))
    return out


# speedup vs baseline: 23.7233x; 1.3968x over previous
import functools

import jax
import jax.numpy as jnp
from jax import lax
from jax.experimental import pallas as pl
from jax.experimental.pallas import tpu as pltpu
from jax.experimental.pallas import tpu_sc as plsc

_N = 10000
_E = 320000
_D = 128
_CL = 16
_NC = 2
_NS = 16
_NW = _NC * _NS
_EPW = _E // _NW
_CH = 80
_NCHUNK = _EPW // _CH
_NP = 10240
_RPS = _NP // _NS
_HW = 16

_DH = _D // _NC
_EPS = _E // _NS
_NCHUNK2 = _EPS // _CH

_UNTILED = pltpu.CompilerParams(use_tc_tiling_on_sc=False)


def _mesh():
    return plsc.VectorSubcoreMesh(core_axis_name="c", subcore_axis_name="s")


@functools.partial(
    pl.kernel,
    out_type=jax.ShapeDtypeStruct((_NC, 2, _NP, _HW), jnp.float32),
    mesh=_mesh(),
    compiler_params=_UNTILED,
    scratch_types=[
        pltpu.VMEM((_NCHUNK, _CH), jnp.int32),
        pltpu.VMEM((_NCHUNK, _CH), jnp.int32),
        pltpu.VMEM((_CH, _HW), jnp.float32),
        pltpu.VMEM_SHARED((_NP, _HW), jnp.float32),
        pltpu.VMEM_SHARED((_NP, _HW), jnp.float32),
        pltpu.SemaphoreType.DMA,
    ],
)
def _hist_kernel(src_hbm, dst_hbm, ones_hbm, zeros_hbm, out_hbm,
                 srcb, dstb, onesb, h0, h1, sem):
    cid = lax.axis_index("c")
    sid = lax.axis_index("s")
    wid = cid * _NS + sid
    pltpu.sync_copy(src_hbm.at[wid], srcb)
    pltpu.sync_copy(dst_hbm.at[wid], dstb)
    pltpu.sync_copy(ones_hbm, onesb)
    r0 = sid * _RPS
    pltpu.sync_copy(zeros_hbm, h0.at[pl.ds(r0, _RPS)])
    pltpu.sync_copy(zeros_hbm, h1.at[pl.ds(r0, _RPS)])
    plsc.subcore_barrier()

    def fire(j, carry):
        pltpu.async_copy(onesb, h0.at[srcb.at[j]], sem, add=True)
        pltpu.async_copy(onesb, h1.at[dstb.at[j]], sem, add=True)
        return carry

    lax.fori_loop(0, _NCHUNK, fire, 0)

    def drain(j, carry):
        pltpu.make_async_copy(onesb, h0.at[srcb.at[j]], sem).wait()
        pltpu.make_async_copy(onesb, h1.at[dstb.at[j]], sem).wait()
        return carry

    lax.fori_loop(0, _NCHUNK, drain, 0)
    plsc.subcore_barrier()
    pltpu.sync_copy(h0.at[pl.ds(r0, _RPS)], out_hbm.at[cid, 0, pl.ds(r0, _RPS)])
    pltpu.sync_copy(h1.at[pl.ds(r0, _RPS)], out_hbm.at[cid, 1, pl.ds(r0, _RPS)])


def _scatter_body(z_view, src_hbm, dst_hbm, zeros_hbm, out_hbm,
                  srcb, dstb, rows0, rows1, acc, sg0, sg1, ss0, ss1,
                  idx_row, cid, sid, nchunk):
    pltpu.sync_copy(src_hbm.at[idx_row], srcb)
    pltpu.sync_copy(dst_hbm.at[idx_row], dstb)
    r0 = sid * _RPS
    pltpu.sync_copy(zeros_hbm, acc.at[pl.ds(r0, _RPS)])
    plsc.subcore_barrier()

    pltpu.async_copy(z_view.at[srcb.at[0]], rows0, sg0)

    def chunk(j, rows_cur, sg_cur, ss_cur, rows_nxt, sg_nxt, ss_nxt):
        @pl.when(j + 1 < nchunk)
        def _start_next():
            @pl.when(j >= 1)
            def _wait_prev_scatter():
                pltpu.make_async_copy(
                    rows_nxt, acc.at[dstb.at[j - 1]], ss_nxt).wait()

            pltpu.async_copy(z_view.at[srcb.at[j + 1]], rows_nxt, sg_nxt)

        pltpu.make_async_copy(z_view.at[srcb.at[j]], rows_cur, sg_cur).wait()
        pltpu.async_copy(rows_cur, acc.at[dstb.at[j]], ss_cur, add=True)

    def body(j, carry):
        @pl.when(j % 2 == 0)
        def _even():
            chunk(j, rows0, sg0, ss0, rows1, sg1, ss1)

        @pl.when(j % 2 == 1)
        def _odd():
            chunk(j, rows1, sg1, ss1, rows0, sg0, ss0)

        return carry

    lax.fori_loop(0, nchunk, body, 0)
    jl, js = nchunk - 1, nchunk - 2
    rl = [rows0, rows1]
    sl = [ss0, ss1]
    pltpu.make_async_copy(rl[jl % 2], acc.at[dstb.at[jl]], sl[jl % 2]).wait()
    pltpu.make_async_copy(rl[js % 2], acc.at[dstb.at[js]], sl[js % 2]).wait()
    plsc.subcore_barrier()
    pltpu.sync_copy(acc.at[pl.ds(r0, _RPS)], out_hbm.at[cid, pl.ds(r0, _RPS)])


@functools.partial(
    pl.kernel,
    out_type=jax.ShapeDtypeStruct((_NC, _NP, _DH), jnp.float32),
    mesh=_mesh(),
    compiler_params=_UNTILED,
    scratch_types=[
        pltpu.VMEM((_NCHUNK2, _CH), jnp.int32),
        pltpu.VMEM((_NCHUNK2, _CH), jnp.int32),
        pltpu.VMEM((_CH, _DH), jnp.float32),
        pltpu.VMEM((_CH, _DH), jnp.float32),
        pltpu.VMEM_SHARED((_NP, _DH), jnp.float32),
        pltpu.SemaphoreType.DMA,
        pltpu.SemaphoreType.DMA,
        pltpu.SemaphoreType.DMA,
        pltpu.SemaphoreType.DMA,
    ],
)
def _scatter_d(z_hbm, src_hbm, dst_hbm, zeros_hbm, out_hbm,
               srcb, dstb, rows0, rows1, acc, sg0, sg1, ss0, ss1):
    cid = lax.axis_index("c")
    sid = lax.axis_index("s")
    _scatter_body(z_hbm.at[cid], src_hbm, dst_hbm, zeros_hbm, out_hbm,
                  srcb, dstb, rows0, rows1, acc, sg0, sg1, ss0, ss1,
                  sid, cid, sid, _NCHUNK2)


@functools.partial(
    pl.kernel,
    out_type=jax.ShapeDtypeStruct((_NC, _NP, _CL), jnp.float32),
    mesh=_mesh(),
    compiler_params=_UNTILED,
    scratch_types=[
        pltpu.VMEM((_NCHUNK, _CH), jnp.int32),
        pltpu.VMEM((_NCHUNK, _CH), jnp.int32),
        pltpu.VMEM((_CH, _CL), jnp.float32),
        pltpu.VMEM((_CH, _CL), jnp.float32),
        pltpu.VMEM_SHARED((_NP, _CL), jnp.float32),
        pltpu.SemaphoreType.DMA,
        pltpu.SemaphoreType.DMA,
        pltpu.SemaphoreType.DMA,
        pltpu.SemaphoreType.DMA,
    ],
)
def _scatter_c(z_hbm, src_hbm, dst_hbm, zeros_hbm, out_hbm,
               srcb, dstb, rows0, rows1, acc, sg0, sg1, ss0, ss1):
    cid = lax.axis_index("c")
    sid = lax.axis_index("s")
    wid = cid * _NS + sid
    _scatter_body(z_hbm, src_hbm, dst_hbm, zeros_hbm, out_hbm,
                  srcb, dstb, rows0, rows1, acc, sg0, sg1, ss0, ss1,
                  wid, cid, sid, _NCHUNK)


def _tc1_body(h_ref, w_ref, dp_ref, z_ref, ac_ref):
    dp = dp_ref[...]
    deg_out = dp[:, 0:1] + dp[:, 2:3]
    deg_in = dp[:, 1:2] + dp[:, 3:4]
    a = lax.rsqrt(jnp.maximum(deg_out, 1.0))
    c = lax.rsqrt(jnp.maximum(deg_in, 1.0))
    y = jnp.dot(h_ref[...], w_ref[...], preferred_element_type=jnp.float32)
    y = y * a
    z_ref[0] = y[:, :_DH]
    z_ref[1] = y[:, _DH:]
    ac_ref[...] = jnp.concatenate([a, c], axis=1)


def _tc2_body(p_ref, ac_ref, b1_ref, w2_ref, z2_ref):
    a = ac_ref[:, 0:1]
    c = ac_ref[:, 1:2]
    agg = jnp.concatenate([p_ref[0, :_N], p_ref[1, :_N]], axis=1)
    h1 = jnp.maximum(agg * c + b1_ref[...], 0.0)
    z2_ref[...] = jnp.dot(h1, w2_ref[...], preferred_element_type=jnp.float32) * a


def _tc3_body(q_ref, ac_ref, b2_ref, o_ref):
    c = ac_ref[:, 1:2]
    o_ref[...] = (q_ref[0, :_N] + q_ref[1, :_N]) * c + b2_ref[...]


def kernel(h, edge_index, W1, b1, W2, b2):
    src32 = edge_index[0].reshape(_NW, _NCHUNK, _CH)
    dst32 = edge_index[1].reshape(_NW, _NCHUNK, _CH)
    src16 = edge_index[0].reshape(_NS, _NCHUNK2, _CH)
    dst16 = edge_index[1].reshape(_NS, _NCHUNK2, _CH)
    ones_rows = jnp.ones((_CH, _HW), jnp.float32)
    zeros_h = jnp.zeros((_RPS, _HW), jnp.float32)
    zeros_d = jnp.zeros((_RPS, _DH), jnp.float32)
    zeros_c = jnp.zeros((_RPS, _CL), jnp.float32)

    hist = _hist_kernel(src32, dst32, ones_rows, zeros_h)
    dp = hist[:, :, :_N, 0].reshape(4, _N).T

    z1s, ac = pl.pallas_call(
        _tc1_body,
        out_shape=[
            jax.ShapeDtypeStruct((_NC, _N, _DH), jnp.float32),
            jax.ShapeDtypeStruct((_N, 2), jnp.float32),
        ],
    )(h, W1, dp)

    p = _scatter_d(z1s, src16, dst16, zeros_d)

    z2 = pl.pallas_call(
        _tc2_body,
        out_shape=jax.ShapeDtypeStruct((_N, _CL), jnp.float32),
    )(p, ac, b1.reshape(1, _D), W2)

    q = _scatter_c(z2, src32, dst32, zeros_c)

    out = pl.pallas_call(
        _tc3_body,
        out_shape=jax.ShapeDtypeStruct((_N, _CL), jnp.float32),
    )(q, ac, b2.reshape(1, _CL))
    return out


# speedup vs baseline: 32.0606x; 1.3514x over previous
import functools

import jax
import jax.numpy as jnp
from jax import lax
from jax.experimental import pallas as pl
from jax.experimental.pallas import tpu as pltpu
from jax.experimental.pallas import tpu_sc as plsc

_N = 10000
_E = 320000
_D = 128
_CL = 16
_NC = 2
_NS = 16
_NW = _NC * _NS
_EPW = _E // _NW
_CH = 80
_NCHUNK = _EPW // _CH
_NP = 10240
_RPS = _NP // _NS
_HW = 16

_DH = _D // _NC
_EPS = _E // _NS
_NCHUNK2 = _EPS // _CH

_UNTILED = pltpu.CompilerParams(use_tc_tiling_on_sc=False)


def _mesh():
    return plsc.VectorSubcoreMesh(core_axis_name="c", subcore_axis_name="s")


@functools.partial(
    pl.kernel,
    out_type=jax.ShapeDtypeStruct((_NC, 2, _NP, _HW), jnp.float32),
    mesh=_mesh(),
    compiler_params=_UNTILED,
    scratch_types=[
        pltpu.VMEM((_NCHUNK, _CH), jnp.int32),
        pltpu.VMEM((_NCHUNK, _CH), jnp.int32),
        pltpu.VMEM((_CH, _HW), jnp.float32),
        pltpu.VMEM_SHARED((_NP, _HW), jnp.float32),
        pltpu.VMEM_SHARED((_NP, _HW), jnp.float32),
        pltpu.SemaphoreType.DMA,
    ],
)
def _hist_kernel(src_hbm, dst_hbm, ones_hbm, zeros_hbm, out_hbm,
                 srcb, dstb, onesb, h0, h1, sem):
    cid = lax.axis_index("c")
    sid = lax.axis_index("s")
    wid = cid * _NS + sid
    pltpu.sync_copy(src_hbm.at[wid], srcb)
    pltpu.sync_copy(dst_hbm.at[wid], dstb)
    pltpu.sync_copy(ones_hbm, onesb)
    r0 = sid * _RPS
    pltpu.sync_copy(zeros_hbm, h0.at[pl.ds(r0, _RPS)])
    pltpu.sync_copy(zeros_hbm, h1.at[pl.ds(r0, _RPS)])
    plsc.subcore_barrier()

    def fire(j, carry):
        pltpu.async_copy(onesb, h0.at[srcb.at[j]], sem, add=True)
        pltpu.async_copy(onesb, h1.at[dstb.at[j]], sem, add=True)
        return carry

    lax.fori_loop(0, _NCHUNK, fire, 0)

    def drain(j, carry):
        pltpu.make_async_copy(onesb, h0.at[srcb.at[j]], sem).wait()
        pltpu.make_async_copy(onesb, h1.at[dstb.at[j]], sem).wait()
        return carry

    lax.fori_loop(0, _NCHUNK, drain, 0)
    plsc.subcore_barrier()
    pltpu.sync_copy(h0.at[pl.ds(r0, _RPS)], out_hbm.at[cid, 0, pl.ds(r0, _RPS)])
    pltpu.sync_copy(h1.at[pl.ds(r0, _RPS)], out_hbm.at[cid, 1, pl.ds(r0, _RPS)])


_NBUF = 8
_DD = 6


def _scatter_body(z_view, src_hbm, dst_hbm, zeros_hbm, out_hbm,
                  srcb, dstb, rows, acc, sg, ss, idx_row, cid, sid, nchunk):
    pltpu.sync_copy(src_hbm.at[idx_row], srcb)
    pltpu.sync_copy(dst_hbm.at[idx_row], dstb)
    r0 = sid * _RPS
    pltpu.sync_copy(zeros_hbm, acc.at[pl.ds(r0, _RPS)])
    plsc.subcore_barrier()

    for b in range(_DD):
        pltpu.async_copy(z_view.at[srcb.at[b]], rows[b], sg[b])

    def step(j, b):
        gb = (b + _DD) % _NBUF

        @pl.when(j + _DD < nchunk)
        def _start_next():
            @pl.when(j + _DD >= _NBUF)
            def _wait_buffer_free():
                pltpu.make_async_copy(
                    rows[gb], acc.at[dstb.at[j + _DD - _NBUF]], ss[gb]).wait()

            pltpu.async_copy(z_view.at[srcb.at[j + _DD]], rows[gb], sg[gb])

        pltpu.make_async_copy(z_view.at[srcb.at[j]], rows[b], sg[b]).wait()
        pltpu.async_copy(rows[b], acc.at[dstb.at[j]], ss[b], add=True)

    def body(j, carry):
        for b in range(_NBUF):
            @pl.when(j % _NBUF == b)
            def _branch(b=b):
                step(j, b)

        return carry

    lax.fori_loop(0, nchunk, body, 0)
    for i in range(nchunk - _NBUF, nchunk):
        b = i % _NBUF
        pltpu.make_async_copy(rows[b], acc.at[dstb.at[i]], ss[b]).wait()
    plsc.subcore_barrier()
    pltpu.sync_copy(acc.at[pl.ds(r0, _RPS)], out_hbm.at[cid, pl.ds(r0, _RPS)])


@functools.partial(
    pl.kernel,
    out_type=jax.ShapeDtypeStruct((_NC, _NP, _DH), jnp.float32),
    mesh=_mesh(),
    compiler_params=_UNTILED,
    scratch_types=[
        pltpu.VMEM((_NCHUNK2, _CH), jnp.int32),
        pltpu.VMEM((_NCHUNK2, _CH), jnp.int32),
        [pltpu.VMEM((_CH, _DH), jnp.float32)] * _NBUF,
        pltpu.VMEM_SHARED((_NP, _DH), jnp.float32),
        [pltpu.SemaphoreType.DMA] * _NBUF,
        [pltpu.SemaphoreType.DMA] * _NBUF,
    ],
)
def _scatter_d(z_hbm, src_hbm, dst_hbm, zeros_hbm, out_hbm,
               srcb, dstb, rows, acc, sg, ss):
    cid = lax.axis_index("c")
    sid = lax.axis_index("s")
    _scatter_body(z_hbm.at[cid], src_hbm, dst_hbm, zeros_hbm, out_hbm,
                  srcb, dstb, rows, acc, sg, ss, sid, cid, sid, _NCHUNK2)


@functools.partial(
    pl.kernel,
    out_type=jax.ShapeDtypeStruct((_NC, _NP, _CL), jnp.float32),
    mesh=_mesh(),
    compiler_params=_UNTILED,
    scratch_types=[
        pltpu.VMEM((_NCHUNK, _CH), jnp.int32),
        pltpu.VMEM((_NCHUNK, _CH), jnp.int32),
        [pltpu.VMEM((_CH, _CL), jnp.float32)] * _NBUF,
        pltpu.VMEM_SHARED((_NP, _CL), jnp.float32),
        [pltpu.SemaphoreType.DMA] * _NBUF,
        [pltpu.SemaphoreType.DMA] * _NBUF,
    ],
)
def _scatter_c(z_hbm, src_hbm, dst_hbm, zeros_hbm, out_hbm,
               srcb, dstb, rows, acc, sg, ss):
    cid = lax.axis_index("c")
    sid = lax.axis_index("s")
    wid = cid * _NS + sid
    _scatter_body(z_hbm, src_hbm, dst_hbm, zeros_hbm, out_hbm,
                  srcb, dstb, rows, acc, sg, ss, wid, cid, sid, _NCHUNK)


def _tc1_body(h_ref, w_ref, dp_ref, z_ref, ac_ref):
    dp = dp_ref[...]
    deg_out = dp[:, 0:1] + dp[:, 2:3]
    deg_in = dp[:, 1:2] + dp[:, 3:4]
    a = lax.rsqrt(jnp.maximum(deg_out, 1.0))
    c = lax.rsqrt(jnp.maximum(deg_in, 1.0))
    y = jnp.dot(h_ref[...], w_ref[...], preferred_element_type=jnp.float32)
    y = y * a
    z_ref[0] = y[:, :_DH]
    z_ref[1] = y[:, _DH:]
    ac_ref[...] = jnp.concatenate([a, c], axis=1)


def _tc2_body(p_ref, ac_ref, b1_ref, w2_ref, z2_ref):
    a = ac_ref[:, 0:1]
    c = ac_ref[:, 1:2]
    agg = jnp.concatenate([p_ref[0, :_N], p_ref[1, :_N]], axis=1)
    h1 = jnp.maximum(agg * c + b1_ref[...], 0.0)
    z2_ref[...] = jnp.dot(h1, w2_ref[...], preferred_element_type=jnp.float32) * a


def _tc3_body(q_ref, ac_ref, b2_ref, o_ref):
    c = ac_ref[:, 1:2]
    o_ref[...] = (q_ref[0, :_N] + q_ref[1, :_N]) * c + b2_ref[...]


def kernel(h, edge_index, W1, b1, W2, b2):
    src32 = edge_index[0].reshape(_NW, _NCHUNK, _CH)
    dst32 = edge_index[1].reshape(_NW, _NCHUNK, _CH)
    src16 = edge_index[0].reshape(_NS, _NCHUNK2, _CH)
    dst16 = edge_index[1].reshape(_NS, _NCHUNK2, _CH)
    ones_rows = jnp.ones((_CH, _HW), jnp.float32)
    zeros_h = jnp.zeros((_RPS, _HW), jnp.float32)
    zeros_d = jnp.zeros((_RPS, _DH), jnp.float32)
    zeros_c = jnp.zeros((_RPS, _CL), jnp.float32)

    hist = _hist_kernel(src32, dst32, ones_rows, zeros_h)
    dp = hist[:, :, :_N, 0].reshape(4, _N).T

    z1s, ac = pl.pallas_call(
        _tc1_body,
        out_shape=[
            jax.ShapeDtypeStruct((_NC, _N, _DH), jnp.float32),
            jax.ShapeDtypeStruct((_N, 2), jnp.float32),
        ],
    )(h, W1, dp)

    p = _scatter_d(z1s, src16, dst16, zeros_d)

    z2 = pl.pallas_call(
        _tc2_body,
        out_shape=jax.ShapeDtypeStruct((_N, _CL), jnp.float32),
    )(p, ac, b1.reshape(1, _D), W2)

    q = _scatter_c(z2, src32, dst32, zeros_c)

    out = pl.pallas_call(
        _tc3_body,
        out_shape=jax.ShapeDtypeStruct((_N, _CL), jnp.float32),
    )(q, ac, b2.reshape(1, _CL))
    return out
